# Initial kernel scaffold; baseline (speedup 1.0000x reference)
import jax
import jax.numpy as jnp
from jax.experimental import pallas as pl


def kernel(x_s, x_t, params, edge_index_s, batch_s, edge_index_t, batch_t):
    raise NotImplementedError("write your pallas kernel here")


# speedup vs baseline: 6.9340x; 6.9340x over previous
import functools

import jax
import jax.numpy as jnp
from jax import lax
from jax.experimental import pallas as pl
from jax.experimental.pallas import tpu as pltpu
from jax.experimental.pallas import tpu_sc as plsc

N = 10000
E = 320000
D = 128
H = 64
NG = 64
NL = 6

NC = 2
NS = 16
CH = 128
CPT = 157
EP = NS * CPT * CH
ROWS_PER_TILE = 640
ACC_ROWS = NS * ROWS_PER_TILE
TRASH = N + 100

BN_ROWS = 2000
NB = (2 * N) // BN_ROWS
NBH = NB // 2


def _mesh():
    return plsc.VectorSubcoreMesh(core_axis_name="c", subcore_axis_name="s",
                                  num_cores=NC, num_subcores=NS)


@functools.cache
def _sc_scatter_kernel():
    return pl.kernel(
        _sc_scatter_body,
        out_type=jax.ShapeDtypeStruct((NC, ACC_ROWS, H), jnp.float32),
        mesh=_mesh(),
        scratch_types=[
            pltpu.VMEM_SHARED((ACC_ROWS, H), jnp.float32),
            pltpu.VMEM((CH,), jnp.int32),
            pltpu.VMEM((CH,), jnp.int32),
            pltpu.VMEM((CH, H), jnp.float32),
            pltpu.SemaphoreType.DMA,
        ],
        compiler_params=pltpu.CompilerParams(use_tc_tiling_on_sc=False),
    )


def _sc_scatter_body(a_hbm, src_hbm, dst_hbm, zeros_hbm, out_hbm,
                     acc, idx_s, idx_d, rows, sem):
    c = lax.axis_index("c")
    s = lax.axis_index("s")
    base = s * ROWS_PER_TILE
    pltpu.sync_copy(zeros_hbm.at[pl.ds(base, ROWS_PER_TILE)],
                    acc.at[pl.ds(base, ROWS_PER_TILE)])
    plsc.subcore_barrier()

    def step(i, carry):
        chunk = s * CPT + i
        pltpu.sync_copy(src_hbm.at[c, chunk], idx_s)
        pltpu.sync_copy(dst_hbm.at[c, chunk], idx_d)
        pltpu.async_copy(a_hbm.at[idx_s], rows, sem).wait()
        pltpu.sync_copy(rows, acc.at[idx_d], add=True)
        return carry

    lax.fori_loop(0, CPT, step, 0)
    plsc.subcore_barrier()
    pltpu.sync_copy(acc.at[pl.ds(base, ROWS_PER_TILE)],
                    out_hbm.at[c, pl.ds(base, ROWS_PER_TILE)])


DEGW = 16


@functools.cache
def _sc_degree_kernel():
    return pl.kernel(
        _sc_degree_body,
        out_type=jax.ShapeDtypeStruct((NC, ACC_ROWS, DEGW), jnp.float32),
        mesh=_mesh(),
        scratch_types=[
            pltpu.VMEM_SHARED((ACC_ROWS, DEGW), jnp.float32),
            pltpu.VMEM((CH,), jnp.int32),
            pltpu.VMEM((CH, DEGW), jnp.float32),
        ],
        compiler_params=pltpu.CompilerParams(use_tc_tiling_on_sc=False),
    )


def _sc_degree_body(dst_hbm, zeros_hbm, ones_hbm, out_hbm, acc, idx_d, rows):
    c = lax.axis_index("c")
    s = lax.axis_index("s")
    base = s * ROWS_PER_TILE
    pltpu.sync_copy(zeros_hbm.at[pl.ds(base, ROWS_PER_TILE)],
                    acc.at[pl.ds(base, ROWS_PER_TILE)])
    pltpu.sync_copy(ones_hbm, rows)
    plsc.subcore_barrier()

    def step(i, carry):
        chunk = s * CPT + i
        pltpu.sync_copy(dst_hbm.at[c, chunk], idx_d)
        pltpu.sync_copy(rows, acc.at[idx_d], add=True)
        return carry

    lax.fori_loop(0, CPT, step, 0)
    plsc.subcore_barrier()
    pltpu.sync_copy(acc.at[pl.ds(base, ROWS_PER_TILE)],
                    out_hbm.at[c, pl.ds(base, ROWS_PER_TILE)])


def _mm(x, w):
    return jax.lax.dot_general(x, w, (((1,), (0,)), ((), ())),
                               preferred_element_type=jnp.float32)


def _lin3(h, w1, b1, w2, b2, w3, b3, deg):
    a = _mm(h, w1[...]) + b1[...]
    b = _mm(h, w2[...]) + b2[...]
    cc = _mm(h, w3[...]) + b3[...]
    return a, cc - deg * b


def _row_spec(bs, cols):
    return pl.BlockSpec((bs, cols), lambda *g: (g[-1], 0))


def _const_spec(shape):
    nd = len(shape)
    return pl.BlockSpec(shape, lambda *g: (0,) * nd)


def _tc_first_body(x_ref, w1, b1, w2, b2, w3, b3, deg_ref, a_out, d_out):
    a, d = _lin3(x_ref[...], w1, b1, w2, b2, w3, b3, deg_ref[...])
    a_out[...] = a
    d_out[...] = d


def _tc_first(x, w1, b1, w2, b2, w3, b3, deg):
    return pl.pallas_call(
        _tc_first_body,
        grid=(NB,),
        in_specs=[_row_spec(BN_ROWS, D)] + [_const_spec(s) for s in
                  ((D, H), (1, H), (D, H), (1, H), (D, H), (1, H))]
                 + [_row_spec(BN_ROWS, 1)],
        out_specs=[_row_spec(BN_ROWS, H), _row_spec(BN_ROWS, H)],
        out_shape=[jax.ShapeDtypeStruct((2 * N, H), jnp.float32)] * 2,
    )(x, w1, b1, w2, b2, w3, b3, deg)


def _bn_h(h2, half, gamma, beta, sum_ref, sq_ref):
    m = sum_ref[pl.ds(half, 1), :] * (1.0 / N)
    v = sq_ref[pl.ds(half, 1), :] * (1.0 / N) - m * m
    return jax.nn.relu((h2 - m) / jnp.sqrt(v + 1e-5) * gamma[...] + beta[...])


def _stats_phase(h2, i, half, sum_ref, sq_ref):
    @pl.when(i == 0)
    def _():
        sum_ref[...] = jnp.zeros_like(sum_ref)
        sq_ref[...] = jnp.zeros_like(sq_ref)

    sum_ref[pl.ds(half, 1), :] += jnp.sum(h2, axis=0, keepdims=True)
    sq_ref[pl.ds(half, 1), :] += jnp.sum(h2 * h2, axis=0, keepdims=True)


def _tc_mid_body(agg_ref, d_ref, gamma, beta, w1, b1, w2, b2, w3, b3, deg_ref,
                 a_out, d_out, sum_ref, sq_ref):
    ph = pl.program_id(0)
    i = pl.program_id(1)
    half = i // NBH
    h2 = agg_ref[...] + d_ref[...]

    @pl.when(ph == 0)
    def _():
        _stats_phase(h2, i, half, sum_ref, sq_ref)

    @pl.when(ph == 1)
    def _():
        h = _bn_h(h2, half, gamma, beta, sum_ref, sq_ref)
        a, d = _lin3(h, w1, b1, w2, b2, w3, b3, deg_ref[...])
        a_out[...] = a
        d_out[...] = d


def _tc_mid(agg, d, gamma, beta, w1, b1, w2, b2, w3, b3, deg):
    return pl.pallas_call(
        _tc_mid_body,
        grid=(2, NB),
        in_specs=[_row_spec(BN_ROWS, H), _row_spec(BN_ROWS, H)]
                 + [_const_spec(s) for s in
                    ((1, H), (1, H), (H, H), (1, H), (H, H), (1, H),
                     (H, H), (1, H))]
                 + [_row_spec(BN_ROWS, 1)],
        out_specs=[_row_spec(BN_ROWS, H), _row_spec(BN_ROWS, H)],
        out_shape=[jax.ShapeDtypeStruct((2 * N, H), jnp.float32)] * 2,
        scratch_shapes=[pltpu.VMEM((2, H), jnp.float32),
                        pltpu.VMEM((2, H), jnp.float32)],
    )(agg, d, gamma, beta, w1, b1, w2, b2, w3, b3, deg)


def _tc_final_body(agg_ref, d_ref, gamma, beta, gw1, gb1, gw2, gb2,
                   batch_c, batch_r,
                   ow0, ob0, ow1, ob1, ow2, ob2, ow3, ob3, out_ref,
                   sum_ref, sq_ref, gmax_ref, esum_ref, num_ref):
    ph = pl.program_id(0)
    i = pl.program_id(1)
    half = i // NBH
    h2 = agg_ref[...] + d_ref[...]

    def gate_h():
        h = _bn_h(h2, half, gamma, beta, sum_ref, sq_ref)
        gate = jax.nn.relu(_mm(jax.nn.relu(_mm(h, gw1[...]) + gb1[...]),
                               gw2[...]) + gb2[...])
        return h, gate

    @pl.when(ph == 0)
    def _():
        _stats_phase(h2, i, half, sum_ref, sq_ref)

        @pl.when(i == 0)
        def _():
            gmax_ref[...] = jnp.full_like(gmax_ref, -3.0e38)
            esum_ref[...] = jnp.zeros_like(esum_ref)
            num_ref[...] = jnp.zeros_like(num_ref)

    @pl.when(ph == 1)
    def _():
        _, gate = gate_h()
        onehot = batch_c[...] == lax.broadcasted_iota(
            jnp.int32, (BN_ROWS, 2 * NG), 1)
        bm = jnp.max(jnp.where(onehot, gate, jnp.float32(-3.0e38)),
                     axis=0, keepdims=True)
        gmax_ref[...] = jnp.maximum(gmax_ref[...], bm)

    @pl.when(ph == 2)
    def _():
        h, gate = gate_h()
        onehot = (batch_c[...] == lax.broadcasted_iota(
            jnp.int32, (BN_ROWS, 2 * NG), 1)).astype(jnp.float32)
        gmax_node = jnp.sum(onehot * gmax_ref[...], axis=1, keepdims=True)
        e = jnp.exp(gate - gmax_node)
        onehot_g = (batch_r[0, :1, :] == lax.broadcasted_iota(
            jnp.int32, (2 * NG, BN_ROWS), 0)).astype(jnp.float32)
        esum_ref[...] += _mm(onehot_g, e)
        num_ref[...] += _mm(onehot_g, e * h)

    @pl.when((ph == 3) & (i == 0))
    def _():
        pooled = num_ref[...] / (esum_ref[...] + 1e-16)
        hh = jnp.abs(pooled[:NG] - pooled[NG:])
        hh = jax.nn.relu(_mm(hh, ow0[...]) + ob0[...])
        hh = jax.nn.relu(_mm(hh, ow1[...]) + ob1[...])
        hh = jax.nn.relu(_mm(hh, ow2[...]) + ob2[...])
        out_ref[...] = _mm(hh, ow3[...]) + ob3[...]


def _tc_final(agg, d, gamma, beta, gw1, gb1, gw2, gb2, batch_c, batch_r,
              *mlp):
    return pl.pallas_call(
        _tc_final_body,
        grid=(4, NB),
        in_specs=[_row_spec(BN_ROWS, H), _row_spec(BN_ROWS, H)]
                 + [_const_spec(s) for s in
                    ((1, H), (1, H), (H, 32), (1, 32), (32, 1), (1, 1))]
                 + [_row_spec(BN_ROWS, 1),
                    pl.BlockSpec((1, 8, BN_ROWS), lambda *g: (g[-1], 0, 0))]
                 + [_const_spec(s) for s in
                    ((H, 64), (1, 64), (64, 64), (1, 64), (64, 64), (1, 64),
                     (64, 1), (1, 1))],
        out_specs=pl.BlockSpec((NG, 1), lambda *g: (0, 0)),
        out_shape=jax.ShapeDtypeStruct((NG, 1), jnp.float32),
        scratch_shapes=[pltpu.VMEM((2, H), jnp.float32),
                        pltpu.VMEM((2, H), jnp.float32),
                        pltpu.VMEM((1, 2 * NG), jnp.float32),
                        pltpu.VMEM((2 * NG, 1), jnp.float32),
                        pltpu.VMEM((2 * NG, H), jnp.float32)],
    )(agg, d, gamma, beta, gw1, gb1, gw2, gb2, batch_c, batch_r, *mlp)


def _forward(x_s, x_t, params, edge_index_s, batch_s, edge_index_t, batch_t,
             degree_fn, scatter_fn):
    p = params
    X = jnp.concatenate([x_s, x_t], axis=0)

    deg = degree_fn(edge_index_s, edge_index_t).reshape(2 * N, 1)

    def b2d(name):
        return p[name].reshape(1, -1)

    a, d = _tc_first(
        X, p["conv0_W1"], b2d("conv0_b1"), p["conv0_W2"], b2d("conv0_b2"),
        p["conv0_W3"], b2d("conv0_b3"), deg)

    out = None
    for l in range(NL):
        agg = scatter_fn(a, edge_index_s, edge_index_t)
        if l < NL - 1:
            nl = l + 1
            a, d = _tc_mid(
                agg, d, b2d("conv%d_gamma" % l), b2d("conv%d_beta" % l),
                p["conv%d_W1" % nl], b2d("conv%d_b1" % nl),
                p["conv%d_W2" % nl], b2d("conv%d_b2" % nl),
                p["conv%d_W3" % nl], b2d("conv%d_b3" % nl), deg)
        else:
            batch_c = jnp.concatenate([batch_s, batch_t + NG]).reshape(
                2 * N, 1)
            batch_r = jnp.broadcast_to(
                batch_c.reshape(NB, 1, BN_ROWS), (NB, 8, BN_ROWS))
            out = _tc_final(
                agg, d, b2d("conv%d_gamma" % l), b2d("conv%d_beta" % l),
                p["gate_W1"], b2d("gate_b1"), p["gate_W2"], b2d("gate_b2"),
                batch_c, batch_r,
                p["out_W0"], b2d("out_b0"), p["out_W1"], b2d("out_b1"),
                p["out_W2"], b2d("out_b2"), p["out_W3"], b2d("out_b3"))
    return out


def _edge_arrays(edge_index_s, edge_index_t):
    pad_e = EP - E
    src = jnp.stack([
        jnp.concatenate([edge_index_s[0], jnp.zeros((pad_e,), jnp.int32)]),
        jnp.concatenate([edge_index_t[0] + N, jnp.zeros((pad_e,), jnp.int32)]),
    ]).reshape(NC, NS * CPT, CH)
    dst = jnp.stack([
        jnp.concatenate([edge_index_s[1],
                         jnp.full((pad_e,), TRASH, jnp.int32)]),
        jnp.concatenate([edge_index_t[1],
                         jnp.full((pad_e,), TRASH, jnp.int32)]),
    ]).reshape(NC, NS * CPT, CH)
    return src, dst


def kernel(x_s, x_t, params, edge_index_s, batch_s, edge_index_t, batch_t):
    src, dst = _edge_arrays(edge_index_s, edge_index_t)
    zeros_h = jnp.zeros((ACC_ROWS, H), jnp.float32)
    zeros_d = jnp.zeros((ACC_ROWS, DEGW), jnp.float32)
    ones_d = jnp.ones((CH, DEGW), jnp.float32)

    def degree_fn(eis, eit):
        return _sc_degree_kernel()(dst, zeros_d, ones_d)[:, :N, 0]

    def scatter_fn(a, eis, eit):
        return _sc_scatter_kernel()(a, src, dst, zeros_h)[:, :N, :].reshape(
            2 * N, H)

    return _forward(x_s, x_t, params, edge_index_s, batch_s,
                    edge_index_t, batch_t, degree_fn, scatter_fn)


# speedup vs baseline: 8.6746x; 1.2510x over previous
import functools

import jax
import jax.numpy as jnp
from jax import lax
from jax.experimental import pallas as pl
from jax.experimental.pallas import tpu as pltpu
from jax.experimental.pallas import tpu_sc as plsc

N = 10000
E = 320000
D = 128
H = 64
NG = 64
NL = 6

NC = 2
NS = 16
CH = 128
CPT = 158
EP = NS * CPT * CH
ROWS_PER_TILE = 640
ACC_ROWS = NS * ROWS_PER_TILE
TRASH = N + 100

BN_ROWS = 2000
NB = (2 * N) // BN_ROWS
NBH = NB // 2


def _mesh():
    return plsc.VectorSubcoreMesh(core_axis_name="c", subcore_axis_name="s",
                                  num_cores=NC, num_subcores=NS)


@functools.cache
def _sc_scatter_kernel():
    return pl.kernel(
        _sc_scatter_body,
        out_type=jax.ShapeDtypeStruct((NC, ACC_ROWS, H), jnp.float32),
        mesh=_mesh(),
        scratch_types=[
            pltpu.VMEM_SHARED((ACC_ROWS, H), jnp.float32),
            pltpu.VMEM((CPT, CH), jnp.int32),
            pltpu.VMEM((CPT, CH), jnp.int32),
            pltpu.VMEM((2, CH, H), jnp.float32),
            pltpu.SemaphoreType.DMA,
            pltpu.SemaphoreType.DMA,
        ],
        compiler_params=pltpu.CompilerParams(use_tc_tiling_on_sc=False),
    )


def _sc_scatter_body(a_hbm, src_hbm, dst_hbm, zeros_hbm, out_hbm,
                     acc, sidx, didx, rows, sem_g, sem_s):
    c = lax.axis_index("c")
    s = lax.axis_index("s")
    base = s * ROWS_PER_TILE
    pltpu.sync_copy(zeros_hbm.at[pl.ds(base, ROWS_PER_TILE)],
                    acc.at[pl.ds(base, ROWS_PER_TILE)])
    pltpu.sync_copy(src_hbm.at[c, pl.ds(s * CPT, CPT)], sidx)
    pltpu.sync_copy(dst_hbm.at[c, pl.ds(s * CPT, CPT)], didx)
    plsc.subcore_barrier()

    def g_start(i, b):
        pltpu.async_copy(a_hbm.at[sidx.at[i]], rows.at[b], sem_g)

    def g_wait(b):
        pltpu.make_async_copy(a_hbm.at[sidx.at[0]], rows.at[b], sem_g).wait()

    def s_start(i, b):
        pltpu.async_copy(rows.at[b], acc.at[didx.at[i]], sem_s, add=True)

    def s_wait(i, b):
        pltpu.make_async_copy(rows.at[b], acc.at[didx.at[i]], sem_s).wait()

    npairs = CPT // 2
    g_start(0, 0)

    def pair(j, carry):
        i0 = 2 * j
        i1 = i0 + 1
        g_wait(0)

        @pl.when(j > 0)
        def _():
            s_wait(i1 - 2, 1)

        g_start(i1, 1)
        s_start(i0, 0)
        g_wait(1)
        s_wait(i0, 0)

        @pl.when(j + 1 < npairs)
        def _():
            g_start(i0 + 2, 0)

        s_start(i1, 1)
        return carry

    lax.fori_loop(0, npairs, pair, 0)
    s_wait(CPT - 1, 1)
    plsc.subcore_barrier()
    pltpu.sync_copy(acc.at[pl.ds(base, ROWS_PER_TILE)],
                    out_hbm.at[c, pl.ds(base, ROWS_PER_TILE)])


DEGW = 16


@functools.cache
def _sc_degree_kernel():
    return pl.kernel(
        _sc_degree_body,
        out_type=jax.ShapeDtypeStruct((NC, ACC_ROWS, DEGW), jnp.float32),
        mesh=_mesh(),
        scratch_types=[
            pltpu.VMEM_SHARED((ACC_ROWS, DEGW), jnp.float32),
            pltpu.VMEM((CPT, CH), jnp.int32),
            pltpu.VMEM((CH, DEGW), jnp.float32),
            pltpu.SemaphoreType.DMA,
        ],
        compiler_params=pltpu.CompilerParams(use_tc_tiling_on_sc=False),
    )


def _sc_degree_body(dst_hbm, zeros_hbm, ones_hbm, out_hbm, acc, didx, rows,
                    sem_s):
    c = lax.axis_index("c")
    s = lax.axis_index("s")
    base = s * ROWS_PER_TILE
    pltpu.sync_copy(zeros_hbm.at[pl.ds(base, ROWS_PER_TILE)],
                    acc.at[pl.ds(base, ROWS_PER_TILE)])
    pltpu.sync_copy(ones_hbm, rows)
    pltpu.sync_copy(dst_hbm.at[c, pl.ds(s * CPT, CPT)], didx)
    plsc.subcore_barrier()

    def step(i, carry):
        pltpu.async_copy(rows, acc.at[didx.at[i]], sem_s, add=True)

        @pl.when(i > 0)
        def _():
            pltpu.make_async_copy(rows, acc.at[didx.at[0]], sem_s).wait()

        return carry

    lax.fori_loop(0, CPT, step, 0)
    pltpu.make_async_copy(rows, acc.at[didx.at[0]], sem_s).wait()
    plsc.subcore_barrier()
    pltpu.sync_copy(acc.at[pl.ds(base, ROWS_PER_TILE)],
                    out_hbm.at[c, pl.ds(base, ROWS_PER_TILE)])


def _mm(x, w):
    return jax.lax.dot_general(x, w, (((1,), (0,)), ((), ())),
                               preferred_element_type=jnp.float32)


def _mm_hi(x, w):
    return jax.lax.dot_general(x, w, (((1,), (0,)), ((), ())),
                               precision=jax.lax.Precision.HIGHEST,
                               preferred_element_type=jnp.float32)


def _lin3(h, w1, b1, w2, b2, w3, b3, deg):
    a = _mm(h, w1[...]) + b1[...]
    b = _mm(h, w2[...]) + b2[...]
    cc = _mm(h, w3[...]) + b3[...]
    return a, cc - deg * b


def _row_spec(bs, cols):
    return pl.BlockSpec((bs, cols), lambda *g: (g[-1], 0))


def _const_spec(shape):
    nd = len(shape)
    return pl.BlockSpec(shape, lambda *g: (0,) * nd)


def _tc_first_body(x_ref, w1, b1, w2, b2, w3, b3, deg_ref, a_out, d_out):
    a, d = _lin3(x_ref[...], w1, b1, w2, b2, w3, b3, deg_ref[...])
    a_out[...] = a
    d_out[...] = d


def _tc_first(x, w1, b1, w2, b2, w3, b3, deg):
    return pl.pallas_call(
        _tc_first_body,
        grid=(NB,),
        in_specs=[_row_spec(BN_ROWS, D)] + [_const_spec(s) for s in
                  ((D, H), (1, H), (D, H), (1, H), (D, H), (1, H))]
                 + [_row_spec(BN_ROWS, 1)],
        out_specs=[_row_spec(BN_ROWS, H), _row_spec(BN_ROWS, H)],
        out_shape=[jax.ShapeDtypeStruct((2 * N, H), jnp.float32)] * 2,
    )(x, w1, b1, w2, b2, w3, b3, deg)


def _bn_h(h2, half, gamma, beta, sum_ref, sq_ref):
    m = sum_ref[pl.ds(half, 1), :] / N
    v = sq_ref[pl.ds(half, 1), :] / N
    return jax.nn.relu((h2 - m) / jnp.sqrt(v + 1e-5) * gamma[...] + beta[...])


def _sum_phase(h2, i, half, sum_ref):
    @pl.when(i == 0)
    def _():
        sum_ref[...] = jnp.zeros_like(sum_ref)

    sum_ref[pl.ds(half, 1), :] += jnp.sum(h2, axis=0, keepdims=True)


def _sq_phase(h2, i, half, sum_ref, sq_ref):
    @pl.when(i == 0)
    def _():
        sq_ref[...] = jnp.zeros_like(sq_ref)

    dev = h2 - sum_ref[pl.ds(half, 1), :] / N
    sq_ref[pl.ds(half, 1), :] += jnp.sum(dev * dev, axis=0, keepdims=True)


def _tc_mid_body(agg_ref, d_ref, gamma, beta, w1, b1, w2, b2, w3, b3, deg_ref,
                 a_out, d_out, sum_ref, sq_ref):
    ph = pl.program_id(0)
    i = pl.program_id(1)
    half = i // NBH
    h2 = agg_ref[...] + d_ref[...]

    @pl.when(ph == 0)
    def _():
        _sum_phase(h2, i, half, sum_ref)

    @pl.when(ph == 1)
    def _():
        _sq_phase(h2, i, half, sum_ref, sq_ref)

    @pl.when(ph == 2)
    def _():
        h = _bn_h(h2, half, gamma, beta, sum_ref, sq_ref)
        a, d = _lin3(h, w1, b1, w2, b2, w3, b3, deg_ref[...])
        a_out[...] = a
        d_out[...] = d


def _tc_mid(agg, d, gamma, beta, w1, b1, w2, b2, w3, b3, deg):
    return pl.pallas_call(
        _tc_mid_body,
        grid=(3, NB),
        in_specs=[_row_spec(BN_ROWS, H), _row_spec(BN_ROWS, H)]
                 + [_const_spec(s) for s in
                    ((1, H), (1, H), (H, H), (1, H), (H, H), (1, H),
                     (H, H), (1, H))]
                 + [_row_spec(BN_ROWS, 1)],
        out_specs=[_row_spec(BN_ROWS, H), _row_spec(BN_ROWS, H)],
        out_shape=[jax.ShapeDtypeStruct((2 * N, H), jnp.float32)] * 2,
        scratch_shapes=[pltpu.VMEM((2, H), jnp.float32),
                        pltpu.VMEM((2, H), jnp.float32)],
    )(agg, d, gamma, beta, w1, b1, w2, b2, w3, b3, deg)


def _tc_final_body(agg_ref, d_ref, gamma, beta, gw1, gb1, gw2, gb2,
                   batch_c, batch_r,
                   ow0, ob0, ow1, ob1, ow2, ob2, ow3, ob3, out_ref,
                   sum_ref, sq_ref, gmax_ref, esum_ref, num_ref):
    ph = pl.program_id(0)
    i = pl.program_id(1)
    half = i // NBH
    h2 = agg_ref[...] + d_ref[...]

    def gate_h():
        h = _bn_h(h2, half, gamma, beta, sum_ref, sq_ref)
        gate = jax.nn.relu(_mm(jax.nn.relu(_mm(h, gw1[...]) + gb1[...]),
                               gw2[...]) + gb2[...])
        return h, gate

    @pl.when(ph == 0)
    def _():
        _sum_phase(h2, i, half, sum_ref)

        @pl.when(i == 0)
        def _():
            gmax_ref[...] = jnp.full_like(gmax_ref, -3.0e38)
            esum_ref[...] = jnp.zeros_like(esum_ref)
            num_ref[...] = jnp.zeros_like(num_ref)

    @pl.when(ph == 1)
    def _():
        _sq_phase(h2, i, half, sum_ref, sq_ref)

    @pl.when(ph == 2)
    def _():
        _, gate = gate_h()
        onehot = batch_c[...] == lax.broadcasted_iota(
            jnp.int32, (BN_ROWS, 2 * NG), 1)
        bm = jnp.max(jnp.where(onehot, gate, jnp.float32(-3.0e38)),
                     axis=0, keepdims=True)
        gmax_ref[...] = jnp.maximum(gmax_ref[...], bm)

    @pl.when(ph == 3)
    def _():
        h, gate = gate_h()
        onehot = (batch_c[...] == lax.broadcasted_iota(
            jnp.int32, (BN_ROWS, 2 * NG), 1)).astype(jnp.float32)
        gmax_node = jnp.sum(onehot * gmax_ref[...], axis=1, keepdims=True)
        e = jnp.exp(gate - gmax_node)
        onehot_g = (batch_r[0, :1, :] == lax.broadcasted_iota(
            jnp.int32, (2 * NG, BN_ROWS), 0)).astype(jnp.float32)
        esum_ref[...] += _mm_hi(onehot_g, e)
        num_ref[...] += _mm_hi(onehot_g, e * h)

    @pl.when((ph == 4) & (i == 0))
    def _():
        pooled = num_ref[...] / (esum_ref[...] + 1e-16)
        hh = jnp.abs(pooled[:NG] - pooled[NG:])
        hh = jax.nn.relu(_mm(hh, ow0[...]) + ob0[...])
        hh = jax.nn.relu(_mm(hh, ow1[...]) + ob1[...])
        hh = jax.nn.relu(_mm(hh, ow2[...]) + ob2[...])
        out_ref[...] = _mm(hh, ow3[...]) + ob3[...]


def _tc_final(agg, d, gamma, beta, gw1, gb1, gw2, gb2, batch_c, batch_r,
              *mlp):
    return pl.pallas_call(
        _tc_final_body,
        grid=(5, NB),
        in_specs=[_row_spec(BN_ROWS, H), _row_spec(BN_ROWS, H)]
                 + [_const_spec(s) for s in
                    ((1, H), (1, H), (H, 32), (1, 32), (32, 1), (1, 1))]
                 + [_row_spec(BN_ROWS, 1),
                    pl.BlockSpec((1, 8, BN_ROWS), lambda *g: (g[-1], 0, 0))]
                 + [_const_spec(s) for s in
                    ((H, 64), (1, 64), (64, 64), (1, 64), (64, 64), (1, 64),
                     (64, 1), (1, 1))],
        out_specs=pl.BlockSpec((NG, 1), lambda *g: (0, 0)),
        out_shape=jax.ShapeDtypeStruct((NG, 1), jnp.float32),
        scratch_shapes=[pltpu.VMEM((2, H), jnp.float32),
                        pltpu.VMEM((2, H), jnp.float32),
                        pltpu.VMEM((1, 2 * NG), jnp.float32),
                        pltpu.VMEM((2 * NG, 1), jnp.float32),
                        pltpu.VMEM((2 * NG, H), jnp.float32)],
    )(agg, d, gamma, beta, gw1, gb1, gw2, gb2, batch_c, batch_r, *mlp)


def _forward(x_s, x_t, params, edge_index_s, batch_s, edge_index_t, batch_t,
             degree_fn, scatter_fn):
    p = params
    X = jnp.concatenate([x_s, x_t], axis=0)

    deg = degree_fn(edge_index_s, edge_index_t).reshape(2 * N, 1)

    def b2d(name):
        return p[name].reshape(1, -1)

    a, d = _tc_first(
        X, p["conv0_W1"], b2d("conv0_b1"), p["conv0_W2"], b2d("conv0_b2"),
        p["conv0_W3"], b2d("conv0_b3"), deg)

    out = None
    for l in range(NL):
        agg = scatter_fn(a, edge_index_s, edge_index_t)
        if l < NL - 1:
            nl = l + 1
            a, d = _tc_mid(
                agg, d, b2d("conv%d_gamma" % l), b2d("conv%d_beta" % l),
                p["conv%d_W1" % nl], b2d("conv%d_b1" % nl),
                p["conv%d_W2" % nl], b2d("conv%d_b2" % nl),
                p["conv%d_W3" % nl], b2d("conv%d_b3" % nl), deg)
        else:
            batch_c = jnp.concatenate([batch_s, batch_t + NG]).reshape(
                2 * N, 1)
            batch_r = jnp.broadcast_to(
                batch_c.reshape(NB, 1, BN_ROWS), (NB, 8, BN_ROWS))
            out = _tc_final(
                agg, d, b2d("conv%d_gamma" % l), b2d("conv%d_beta" % l),
                p["gate_W1"], b2d("gate_b1"), p["gate_W2"], b2d("gate_b2"),
                batch_c, batch_r,
                p["out_W0"], b2d("out_b0"), p["out_W1"], b2d("out_b1"),
                p["out_W2"], b2d("out_b2"), p["out_W3"], b2d("out_b3"))
    return out


def _edge_arrays(edge_index_s, edge_index_t):
    pad_e = EP - E
    src = jnp.stack([
        jnp.concatenate([edge_index_s[0], jnp.zeros((pad_e,), jnp.int32)]),
        jnp.concatenate([edge_index_t[0] + N, jnp.zeros((pad_e,), jnp.int32)]),
    ]).reshape(NC, NS * CPT, CH)
    dst = jnp.stack([
        jnp.concatenate([edge_index_s[1],
                         jnp.full((pad_e,), TRASH, jnp.int32)]),
        jnp.concatenate([edge_index_t[1],
                         jnp.full((pad_e,), TRASH, jnp.int32)]),
    ]).reshape(NC, NS * CPT, CH)
    return src, dst


def kernel(x_s, x_t, params, edge_index_s, batch_s, edge_index_t, batch_t):
    src, dst = _edge_arrays(edge_index_s, edge_index_t)
    zeros_h = jnp.zeros((ACC_ROWS, H), jnp.float32)
    zeros_d = jnp.zeros((ACC_ROWS, DEGW), jnp.float32)
    ones_d = jnp.ones((CH, DEGW), jnp.float32)

    def degree_fn(eis, eit):
        return _sc_degree_kernel()(dst, zeros_d, ones_d)[:, :N, 0]

    def scatter_fn(a, eis, eit):
        return _sc_scatter_kernel()(a, src, dst, zeros_h)[:, :N, :].reshape(
            2 * N, H)

    return _forward(x_s, x_t, params, edge_index_s, batch_s,
                    edge_index_t, batch_t, degree_fn, scatter_fn)
